# baseline (speedup 1.0000x reference)
import jax
import jax.numpy as jnp
from jax.experimental import pallas as pl
from jax.experimental.pallas import tpu as pltpu

_BM = 200
_KCACHE = 8
_VMEM_LIMIT = 128 * 1024 * 1024


def _proj_kernel(x_ref, w_ref, o_ref):
    o_ref[...] = jnp.dot(x_ref[...], w_ref[...],
                         preferred_element_type=jnp.float32
                         ).astype(jnp.bfloat16)


def kernel(x, support, W0, b0, W1, b1, Wp, bp):
    n, d_in = x.shape
    d_h = W0.shape[1]
    d_out = Wp.shape[1]
    bm = _BM
    nb = n // bm
    kc = min(_KCACHE, (nb - 2) // 2)

    b0r = b0.reshape(1, -1)
    b1r = b1.reshape(1, -1)
    bpr = bp.reshape(1, -1)
    wp_top = Wp[:d_h]
    wp_bot = Wp[d_h:]

    g0 = pl.pallas_call(
        _proj_kernel,
        out_shape=jax.ShapeDtypeStruct((n, d_h), jnp.bfloat16),
    )(x, W0)

    def _pi(i):
        inter = jnp.where(i % 2 == 1, kc + i // 2, i // 2 - 1)
        mid = jnp.where(i <= 2 * kc + 1, inter, i - 1)
        return jnp.where(i == 0, nb - 1, mid)

    def _sigma(i):
        inter = jnp.where(i % 2 == 1, kc + i // 2, kc + i // 2 - 1)
        mid = jnp.where(i <= 2 * kc + 1, inter, i - 1)
        return jnp.where(i == 0, nb - 1, mid)

    def body(s_ref, g0_ref, b0_ref, w1_ref, wpt_ref, bp_ref, b1_ref,
             wpb_ref, o_ref, cache_ref, g1f_ref, g1b_ref, p_ref):
        ph = pl.program_id(0)
        i = pl.program_id(1)

        def layer0(s_bf):
            h = jnp.dot(s_bf, g0_ref[...], preferred_element_type=jnp.float32)
            h = jnp.maximum(h + b0_ref[...], 0.0)
            g1f_ref[pl.ds(i * bm, bm), :] = jnp.dot(
                h, w1_ref[...], preferred_element_type=jnp.float32)
            p_ref[i] = (jnp.dot(h, wpt_ref[...],
                                preferred_element_type=jnp.float32)
                        + bp_ref[...]).astype(jnp.bfloat16)

        @pl.when(jnp.logical_and(ph == 0, i < kc))
        def _phase0_cached():
            cache_ref[i] = s_ref[...].astype(jnp.bfloat16)
            layer0(cache_ref[i])

        @pl.when(jnp.logical_and(ph == 0, i >= kc))
        def _phase0_streamed():
            layer0(s_ref[...].astype(jnp.bfloat16))

        @pl.when(jnp.logical_and(ph == 1, i == 0))
        def _cast_g1():
            g1b_ref[...] = g1f_ref[...].astype(jnp.bfloat16)

        @pl.when(ph == 1)
        def _phase1():
            blk = _pi(i)
            cached = jnp.logical_and(
                jnp.logical_and(i % 2 == 0, i >= 2), i <= 2 * kc)

            def emit(s_bf):
                h = jnp.dot(s_bf, g1b_ref[...],
                            preferred_element_type=jnp.float32)
                h = jnp.maximum(h + b1_ref[...], 0.0)
                o_ref[...] = (
                    p_ref[blk].astype(jnp.float32)
                    + jnp.dot(h, wpb_ref[...],
                              preferred_element_type=jnp.float32))

            @pl.when(cached)
            def _():
                emit(cache_ref[blk])

            @pl.when(jnp.logical_not(cached))
            def _():
                emit(s_ref[...].astype(jnp.bfloat16))

    const = lambda r, c: pl.BlockSpec((r, c), lambda p, i: (0, 0))

    out = pl.pallas_call(
        body,
        grid=(2, nb),
        in_specs=[
            pl.BlockSpec((bm, n),
                         lambda p, i: (jnp.where(p == 0, i, _sigma(i)), 0)),
            const(n, d_h),
            const(1, d_h),
            const(d_h, d_h),
            const(d_h, d_out),
            const(1, d_out),
            const(1, d_h),
            const(d_h, d_out),
        ],
        out_specs=pl.BlockSpec(
            (bm, d_out), lambda p, i: (jnp.where(p == 0, 0, _pi(i)), 0)),
        out_shape=jax.ShapeDtypeStruct((n, d_out), jnp.float32),
        scratch_shapes=[
            pltpu.VMEM((kc, bm, n), jnp.bfloat16),
            pltpu.VMEM((n, d_h), jnp.float32),
            pltpu.VMEM((n, d_h), jnp.bfloat16),
            pltpu.VMEM((nb, bm, d_out), jnp.bfloat16),
        ],
        compiler_params=pltpu.CompilerParams(
            dimension_semantics=("arbitrary", "arbitrary"),
            vmem_limit_bytes=_VMEM_LIMIT),
    )(support, g0, b0r, W1, wp_top, bpr, b1r, wp_bot)

    return out


# speedup vs baseline: 1.0658x; 1.0658x over previous
import jax
import jax.numpy as jnp
from jax.experimental import pallas as pl
from jax.experimental.pallas import tpu as pltpu

_BM = 400
_KCACHE = 3
_VMEM_LIMIT = 128 * 1024 * 1024


def _proj_kernel(x_ref, w_ref, o_ref):
    o_ref[...] = jnp.dot(x_ref[...], w_ref[...],
                         preferred_element_type=jnp.float32
                         ).astype(jnp.bfloat16)


def kernel(x, support, W0, b0, W1, b1, Wp, bp):
    n, d_in = x.shape
    d_h = W0.shape[1]
    d_out = Wp.shape[1]
    bm = _BM
    nb = n // bm
    kc = min(_KCACHE, (nb - 2) // 2)

    b0r = b0.reshape(1, -1)
    b1r = b1.reshape(1, -1)
    bpr = bp.reshape(1, -1)
    wp_top = Wp[:d_h]
    wp_bot = Wp[d_h:]

    g0 = pl.pallas_call(
        _proj_kernel,
        out_shape=jax.ShapeDtypeStruct((n, d_h), jnp.bfloat16),
    )(x, W0)

    def _pi(i):
        inter = jnp.where(i % 2 == 1, kc + i // 2, i // 2 - 1)
        mid = jnp.where(i <= 2 * kc + 1, inter, i - 1)
        return jnp.where(i == 0, nb - 1, mid)

    def _sigma(i):
        inter = jnp.where(i % 2 == 1, kc + i // 2, kc + i // 2 - 1)
        mid = jnp.where(i <= 2 * kc + 1, inter, i - 1)
        return jnp.where(i == 0, nb - 1, mid)

    def body(s_ref, g0_ref, b0_ref, w1_ref, wpt_ref, bp_ref, b1_ref,
             wpb_ref, o_ref, cache_ref, g1_ref, p_ref):
        ph = pl.program_id(0)
        i = pl.program_id(1)

        def layer0(s_bf):
            h = jnp.dot(s_bf, g0_ref[...], preferred_element_type=jnp.float32)
            h = jnp.maximum(h + b0_ref[...], 0.0)
            rows = pl.ds(i * bm, bm)
            g1_ref[rows, :] = jnp.dot(
                h, w1_ref[...], preferred_element_type=jnp.float32
            ).astype(jnp.bfloat16)
            p_ref[rows, :] = (jnp.dot(h, wpt_ref[...],
                                      preferred_element_type=jnp.float32)
                              + bp_ref[...]).astype(jnp.bfloat16)

        @pl.when(jnp.logical_and(ph == 0, i < kc))
        def _phase0_cached():
            cache_ref[pl.ds(i * bm, bm), :] = s_ref[...].astype(jnp.bfloat16)
            layer0(cache_ref[pl.ds(i * bm, bm), :])

        @pl.when(jnp.logical_and(ph == 0, i >= kc))
        def _phase0_streamed():
            layer0(s_ref[...].astype(jnp.bfloat16))

        @pl.when(ph == 1)
        def _phase1():
            blk = _pi(i)
            cached = jnp.logical_and(
                jnp.logical_and(i % 2 == 0, i >= 2), i <= 2 * kc)

            def emit(s_bf):
                h = jnp.dot(s_bf, g1_ref[...],
                            preferred_element_type=jnp.float32)
                h = jnp.maximum(h + b1_ref[...], 0.0)
                o_ref[...] = (
                    p_ref[pl.ds(blk * bm, bm), :].astype(jnp.float32)
                    + jnp.dot(h, wpb_ref[...],
                              preferred_element_type=jnp.float32))

            @pl.when(cached)
            def _():
                emit(cache_ref[pl.ds(blk * bm, bm), :])

            @pl.when(jnp.logical_not(cached))
            def _():
                emit(s_ref[...].astype(jnp.bfloat16))

    const = lambda r, c: pl.BlockSpec((r, c), lambda p, i: (0, 0))

    out = pl.pallas_call(
        body,
        grid=(2, nb),
        in_specs=[
            pl.BlockSpec((bm, n),
                         lambda p, i: (jnp.where(p == 0, i, _sigma(i)), 0)),
            const(n, d_h),
            const(1, d_h),
            const(d_h, d_h),
            const(d_h, d_out),
            const(1, d_out),
            const(1, d_h),
            const(d_h, d_out),
        ],
        out_specs=pl.BlockSpec(
            (bm, d_out), lambda p, i: (jnp.where(p == 0, 0, _pi(i)), 0)),
        out_shape=jax.ShapeDtypeStruct((n, d_out), jnp.float32),
        scratch_shapes=[
            pltpu.VMEM((kc * bm, n), jnp.bfloat16),
            pltpu.VMEM((n, d_h), jnp.bfloat16),
            pltpu.VMEM((n, d_out), jnp.bfloat16),
        ],
        compiler_params=pltpu.CompilerParams(
            dimension_semantics=("arbitrary", "arbitrary"),
            internal_scratch_in_bytes=4 * 1024 * 1024,
            vmem_limit_bytes=_VMEM_LIMIT),
    )(support, g0, b0r, W1, wp_top, bpr, b1r, wp_bot)

    return out
